# baseline (speedup 1.0000x reference)
import functools

import jax
import jax.numpy as jnp
from jax import lax
from jax.experimental import pallas as pl
from jax.experimental.pallas import tpu as pltpu
from jax.experimental.pallas import tpu_sc as plsc

D = 32
CHUNK = 128
LOOKAHEAD = 4
NBUF = 8


@functools.lru_cache(maxsize=None)
def _build(n_total: int, nw: int):
    per_w = n_total // nw
    n_chunks = per_w // CHUNK
    assert n_chunks % NBUF == 0 and LOOKAHEAD < NBUF <= n_chunks
    mesh = plsc.VectorSubcoreMesh(core_axis_name="c", subcore_axis_name="s")

    @functools.partial(
        pl.kernel,
        mesh=mesh,
        out_type=jax.ShapeDtypeStruct((n_total, D), jnp.float32),
        scratch_types=[
            pltpu.VMEM((n_chunks, CHUNK), jnp.int32),
            *[pltpu.VMEM((CHUNK, D), jnp.float32) for _ in range(NBUF)],
            *[pltpu.SemaphoreType.DMA for _ in range(2 * NBUF)],
        ],
        compiler_params=pltpu.CompilerParams(use_tc_tiling_on_sc=False),
    )
    def embed_kernel(idx_hbm, table_hbm, out_hbm, idx_v, *rest):
        rows = rest[:NBUF]
        g_sems = rest[NBUF : 2 * NBUF]
        w_sems = rest[2 * NBUF : 3 * NBUF]
        wid = lax.axis_index("s") * 2 + lax.axis_index("c")
        base = wid * per_w

        pltpu.sync_copy(idx_hbm.at[wid], idx_v)

        def gstart(j, b):
            pltpu.async_copy(table_hbm.at[idx_v.at[j]], rows[b], g_sems[b])

        def gwait(j, b):
            pltpu.make_async_copy(
                table_hbm.at[idx_v.at[j]], rows[b], g_sems[b]
            ).wait()

        def out_at(j):
            return out_hbm.at[pl.ds(base + j * CHUNK, CHUNK)]

        def wstart(j, b):
            pltpu.async_copy(rows[b], out_at(j), w_sems[b])

        def wwait(j, b):
            pltpu.make_async_copy(rows[b], out_at(j), w_sems[b]).wait()

        L, B = LOOKAHEAD, NBUF

        for j in range(L):
            gstart(j, j % B)
        for j in range(B - L):
            gwait(j, j % B)
            wstart(j, j % B)
            gstart(j + L, (j + L) % B)

        def body(g, carry):
            for u in range(B):
                j = g * B + (B - L) + u
                b = (B - L + u) % B
                gwait(j, b)
                wstart(j, b)
                lb = u % B
                wwait(j + L - B, lb)
                gstart(j + L, lb)
            return carry

        lax.fori_loop(0, (n_chunks - B) // B, body, 0)

        for j in range(n_chunks - L, n_chunks):
            gwait(j, j % B)
            wstart(j, j % B)
        for j in range(n_chunks - B, n_chunks):
            wwait(j, j % B)

    return embed_kernel


def kernel(x, table):
    batch, n_fields = x.shape
    n_total = batch * n_fields
    info = plsc.get_sparse_core_info()
    nw = info.num_cores * info.num_subcores
    per_w = n_total // nw
    idx = x.reshape(nw, per_w // CHUNK, CHUNK).astype(jnp.int32)
    out = _build(n_total, nw)(idx, table)
    return out.reshape(batch, n_fields, table.shape[1])
